# baseline (speedup 1.0000x reference)
import functools

import jax
import jax.numpy as jnp
from jax import lax
from jax.experimental import pallas as pl
from jax.experimental.pallas import tpu as pltpu
from jax.experimental.pallas import tpu_sc as plsc

N = 262144
D = 64
K = 512
COMMITMENT_COST = 0.25
EPS = 1e-12

B = 2048
NB = N // B

_SC_INFO = plsc.get_sparse_core_info()
NC = _SC_INFO.num_cores
NS = _SC_INFO.num_subcores
NW = NC * NS
BPW = N // NW
C = 512
NCHUNK = BPW // C


def _vq_body(xt_ref, cb_ref, cbn_out_ref, idx_ref, loss_ref, cbn_ref):
    step = pl.program_id(0)

    @pl.when(step == 0)
    def _():
        cb = cb_ref[...]
        cb_n = jnp.sqrt(jnp.sum(cb * cb, axis=1, keepdims=True))
        cbn = cb / jnp.maximum(cb_n, EPS)
        cbn_ref[...] = cbn
        cbn_out_ref[...] = cbn
        loss_ref[0, 0] = 0.0

    xt = xt_ref[...]
    norm = jnp.sqrt(jnp.sum(xt * xt, axis=0, keepdims=True))
    xnt = xt / jnp.maximum(norm, EPS)
    xsq = jnp.sum(xnt * xnt, axis=0, keepdims=True)

    g = lax.dot_general(
        cbn_ref[...], xnt, (((1,), (0,)), ((), ())),
        preferred_element_type=jnp.float32)
    half = K // 2
    take = g[half:] > g[:half]
    v = jnp.where(take, g[half:], g[:half])
    off = jnp.where(take, half, 0).astype(jnp.int32)
    half //= 2
    while half >= 8:
        take = v[half:] > v[:half]
        v = jnp.where(take, v[half:], v[:half])
        off = jnp.where(take, off[half:] + half, off[:half])
        half //= 2
    gmax = jnp.max(v, axis=0, keepdims=True)
    idx = jnp.min(jnp.where(v == gmax, off, K), axis=0)
    idx_ref[0, 0, :] = idx

    loss_ref[0, 0] += jnp.sum(xsq) + float(B) - 2.0 * jnp.sum(gmax)


_sc_mesh = plsc.VectorSubcoreMesh(core_axis_name="c", subcore_axis_name="s")


@functools.partial(
    pl.kernel,
    mesh=_sc_mesh,
    out_type=jax.ShapeDtypeStruct((N, D), jnp.float32),
    compiler_params=pltpu.CompilerParams(use_tc_tiling_on_sc=False),
    scratch_types=[
        pltpu.VMEM((C,), jnp.int32),
        pltpu.VMEM((C,), jnp.int32),
        pltpu.VMEM((C, D), jnp.float32),
        pltpu.VMEM((C, D), jnp.float32),
        pltpu.SemaphoreType.DMA,
        pltpu.SemaphoreType.DMA,
        pltpu.SemaphoreType.DMA,
        pltpu.SemaphoreType.DMA,
    ],
)
def _sc_gather(cbn_hbm, idx_hbm, out_hbm, idx0, idx1, rows0, rows1,
               gsem0, gsem1, osem0, osem1):
    wid = lax.axis_index("s") * NC + lax.axis_index("c")
    base0 = wid * BPW
    idxb = (idx0, idx1)
    rowsb = (rows0, rows1)
    gsems = (gsem0, gsem1)
    osems = (osem0, osem1)
    handles_g = {}
    handles_o = {}

    def start(c):
        b = c & 1
        if c >= 2:
            handles_o[c - 2].wait()
        pltpu.sync_copy(idx_hbm.at[pl.ds(base0 + c * C, C)], idxb[b])
        handles_g[c] = pltpu.async_copy(cbn_hbm.at[idxb[b]], rowsb[b], gsems[b])

    start(0)
    for c in range(NCHUNK):
        if c + 1 < NCHUNK:
            start(c + 1)
        handles_g[c].wait()
        handles_o[c] = pltpu.async_copy(
            rowsb[c & 1], out_hbm.at[pl.ds(base0 + c * C, C)], osems[c & 1])
    handles_o[NCHUNK - 2].wait()
    handles_o[NCHUNK - 1].wait()


@jax.jit
def kernel(x, codebook):
    cbn, idx3, loss_sum = pl.pallas_call(
        _vq_body,
        grid=(NB,),
        in_specs=[
            pl.BlockSpec((D, B), lambda i: (0, i)),
            pl.BlockSpec((K, D), lambda i: (0, 0)),
        ],
        out_specs=[
            pl.BlockSpec((K, D), lambda i: (0, 0)),
            pl.BlockSpec((1, 1, B), lambda i: (i, 0, 0)),
            pl.BlockSpec((1, 1), lambda i: (0, 0), memory_space=pltpu.SMEM),
        ],
        out_shape=[
            jax.ShapeDtypeStruct((K, D), jnp.float32),
            jax.ShapeDtypeStruct((NB, 1, B), jnp.int32),
            jax.ShapeDtypeStruct((1, 1), jnp.float32),
        ],
        scratch_shapes=[pltpu.VMEM((K, D), jnp.float32)],
    )(x.T, codebook)
    idx = idx3.reshape(N)
    q = _sc_gather(cbn, idx)
    loss = (loss_sum * ((1.0 + COMMITMENT_COST) / (N * D))).reshape(())
    return q, loss, idx


# speedup vs baseline: 1.3280x; 1.3280x over previous
import functools

import jax
import jax.numpy as jnp
from jax import lax
from jax.experimental import pallas as pl
from jax.experimental.pallas import tpu as pltpu
from jax.experimental.pallas import tpu_sc as plsc

N = 262144
D = 64
K = 512
COMMITMENT_COST = 0.25
EPS = 1e-12

B = 2048
NB = N // B

_SC_INFO = plsc.get_sparse_core_info()
NC = _SC_INFO.num_cores
NS = _SC_INFO.num_subcores
NW = NC * NS
BPW = N // NW
C = 1024
NCHUNK = BPW // C


def _vq_body(xt_ref, cb_ref, cbn_out_ref, idx_ref, loss_ref, cbn_ref):
    step = pl.program_id(0)

    @pl.when(step == 0)
    def _():
        cb = cb_ref[...]
        cb_n = jnp.sqrt(jnp.sum(cb * cb, axis=1, keepdims=True))
        cbn = cb / jnp.maximum(cb_n, EPS)
        cbn_ref[...] = cbn
        cbn_out_ref[...] = cbn
        loss_ref[0, 0] = 0.0

    xt = xt_ref[...]
    norm = jnp.sqrt(jnp.sum(xt * xt, axis=0, keepdims=True))
    xnt = xt / jnp.maximum(norm, EPS)
    xsq = jnp.sum(xnt * xnt, axis=0, keepdims=True)

    g = lax.dot_general(
        cbn_ref[...], xnt, (((1,), (0,)), ((), ())),
        preferred_element_type=jnp.float32)
    gmax = jnp.max(g, axis=0, keepdims=True)
    row = lax.broadcasted_iota(jnp.int32, (K, B), 0)
    idx = jnp.min(jnp.where(g == gmax, row, K), axis=0)
    idx_ref[0, 0, :] = idx

    loss_ref[0, 0] += jnp.sum(xsq) + float(B) - 2.0 * jnp.sum(gmax)


_sc_mesh = plsc.VectorSubcoreMesh(core_axis_name="c", subcore_axis_name="s")


@functools.partial(
    pl.kernel,
    mesh=_sc_mesh,
    out_type=jax.ShapeDtypeStruct((N, D), jnp.float32),
    compiler_params=pltpu.CompilerParams(use_tc_tiling_on_sc=False),
    scratch_types=[
        pltpu.VMEM((C,), jnp.int32),
        pltpu.VMEM((C, D), jnp.float32),
        pltpu.SemaphoreType.DMA,
    ],
)
def _sc_gather(cbn_hbm, idx_hbm, out_hbm, idx_v, rows_v, sem):
    wid = lax.axis_index("s") * NC + lax.axis_index("c")
    base0 = wid * BPW
    for c in range(NCHUNK):
        base = base0 + c * C
        pltpu.sync_copy(idx_hbm.at[pl.ds(base, C)], idx_v)
        pltpu.async_copy(cbn_hbm.at[idx_v], rows_v, sem).wait()
        pltpu.sync_copy(rows_v, out_hbm.at[pl.ds(base, C)])


@jax.jit
def kernel(x, codebook):
    cbn, idx3, loss_sum = pl.pallas_call(
        _vq_body,
        grid=(NB,),
        in_specs=[
            pl.BlockSpec((D, B), lambda i: (0, i)),
            pl.BlockSpec((K, D), lambda i: (0, 0)),
        ],
        out_specs=[
            pl.BlockSpec((K, D), lambda i: (0, 0)),
            pl.BlockSpec((1, 1, B), lambda i: (i, 0, 0)),
            pl.BlockSpec((1, 1), lambda i: (0, 0), memory_space=pltpu.SMEM),
        ],
        out_shape=[
            jax.ShapeDtypeStruct((K, D), jnp.float32),
            jax.ShapeDtypeStruct((NB, 1, B), jnp.int32),
            jax.ShapeDtypeStruct((1, 1), jnp.float32),
        ],
        scratch_shapes=[pltpu.VMEM((K, D), jnp.float32)],
    )(x.T, codebook)
    idx = idx3.reshape(N)
    q = _sc_gather(cbn, idx)
    loss = (loss_sum * ((1.0 + COMMITMENT_COST) / (N * D))).reshape(())
    return q, loss, idx
